# Initial kernel scaffold; baseline (speedup 1.0000x reference)
import jax
import jax.numpy as jnp
from jax.experimental import pallas as pl


def kernel(X, subject_idx, W):
    raise NotImplementedError("write your pallas kernel here")


# speedup vs baseline: 1.0061x; 1.0061x over previous
import jax
import jax.numpy as jnp
from jax.experimental import pallas as pl
from jax.experimental.pallas import tpu as pltpu


def _body(idx_ref, w_ref, x_ref, o_ref):
    w = w_ref[0].astype(jnp.bfloat16)
    x = x_ref[0].astype(jnp.bfloat16)
    o_ref[0] = jax.lax.dot_general(
        w, x,
        dimension_numbers=(((1,), (0,)), ((), ())),
        preferred_element_type=jnp.float32,
    )


def kernel(X, subject_idx, W):
    N, C, T = X.shape

    grid_spec = pltpu.PrefetchScalarGridSpec(
        num_scalar_prefetch=1,
        grid=(N,),
        in_specs=[
            pl.BlockSpec((1, C, C), lambda n, idx: (idx[n], 0, 0)),
            pl.BlockSpec((1, C, T), lambda n, idx: (n, 0, 0)),
        ],
        out_specs=pl.BlockSpec((1, C, T), lambda n, idx: (n, 0, 0)),
    )
    return pl.pallas_call(
        _body,
        grid_spec=grid_spec,
        out_shape=jax.ShapeDtypeStruct((N, C, T), jnp.float32),
    )(subject_idx, W, X)


# speedup vs baseline: 1.0519x; 1.0456x over previous
import jax
import jax.numpy as jnp
from jax.experimental import pallas as pl
from jax.experimental.pallas import tpu as pltpu


def _body(idx_ref, w_ref, x_ref, o_ref):
    n = pl.program_id(0)
    s = idx_ref[n]
    w = w_ref[s].astype(jnp.bfloat16)
    x = x_ref[0].astype(jnp.bfloat16)
    o_ref[0] = jax.lax.dot_general(
        w, x,
        dimension_numbers=(((1,), (0,)), ((), ())),
        preferred_element_type=jnp.float32,
    )


def kernel(X, subject_idx, W):
    N, C, T = X.shape
    S = W.shape[0]

    grid_spec = pltpu.PrefetchScalarGridSpec(
        num_scalar_prefetch=1,
        grid=(N,),
        in_specs=[
            pl.BlockSpec((S, C, C), lambda n, idx: (0, 0, 0)),
            pl.BlockSpec((1, C, T), lambda n, idx: (n, 0, 0)),
        ],
        out_specs=pl.BlockSpec((1, C, T), lambda n, idx: (n, 0, 0)),
    )
    return pl.pallas_call(
        _body,
        grid_spec=grid_spec,
        out_shape=jax.ShapeDtypeStruct((N, C, T), jnp.float32),
    )(subject_idx, W, X)


# speedup vs baseline: 1.3544x; 1.2875x over previous
import jax
import jax.numpy as jnp
from jax.experimental import pallas as pl
from jax.experimental.pallas import tpu as pltpu


_BN = 4


def _body(idx_ref, w_ref, x_ref, o_ref):
    g = pl.program_id(0)
    for j in range(_BN):
        s = idx_ref[g * _BN + j]
        w = w_ref[s].astype(jnp.bfloat16)
        x = x_ref[j].astype(jnp.bfloat16)
        o_ref[j] = jax.lax.dot_general(
            w, x,
            dimension_numbers=(((1,), (0,)), ((), ())),
            preferred_element_type=jnp.float32,
        )


def kernel(X, subject_idx, W):
    N, C, T = X.shape
    S = W.shape[0]

    grid_spec = pltpu.PrefetchScalarGridSpec(
        num_scalar_prefetch=1,
        grid=(N // _BN,),
        in_specs=[
            pl.BlockSpec((S, C, C), lambda n, idx: (0, 0, 0)),
            pl.BlockSpec((_BN, C, T), lambda n, idx: (n, 0, 0)),
        ],
        out_specs=pl.BlockSpec((_BN, C, T), lambda n, idx: (n, 0, 0)),
    )
    return pl.pallas_call(
        _body,
        grid_spec=grid_spec,
        out_shape=jax.ShapeDtypeStruct((N, C, T), jnp.float32),
    )(subject_idx, W, X)


# speedup vs baseline: 1.4147x; 1.0445x over previous
import jax
import jax.numpy as jnp
from jax.experimental import pallas as pl
from jax.experimental.pallas import tpu as pltpu


_BN = 8


def _body(idx_ref, w_ref, x_ref, o_ref):
    g = pl.program_id(0)
    for j in range(_BN):
        s = idx_ref[g * _BN + j]
        w = w_ref[s].astype(jnp.bfloat16)
        x = x_ref[j].astype(jnp.bfloat16)
        o_ref[j] = jax.lax.dot_general(
            w, x,
            dimension_numbers=(((1,), (0,)), ((), ())),
            preferred_element_type=jnp.float32,
        )


def kernel(X, subject_idx, W):
    N, C, T = X.shape
    S = W.shape[0]

    grid_spec = pltpu.PrefetchScalarGridSpec(
        num_scalar_prefetch=1,
        grid=(N // _BN,),
        in_specs=[
            pl.BlockSpec((S, C, C), lambda n, idx: (0, 0, 0)),
            pl.BlockSpec((_BN, C, T), lambda n, idx: (n, 0, 0)),
        ],
        out_specs=pl.BlockSpec((_BN, C, T), lambda n, idx: (n, 0, 0)),
    )
    return pl.pallas_call(
        _body,
        grid_spec=grid_spec,
        out_shape=jax.ShapeDtypeStruct((N, C, T), jnp.float32),
    )(subject_idx, W, X)


# speedup vs baseline: 1.4277x; 1.0093x over previous
import jax
import jax.numpy as jnp
from jax.experimental import pallas as pl
from jax.experimental.pallas import tpu as pltpu


_BN = 16


def _body(idx_ref, w_ref, x_ref, o_ref):
    g = pl.program_id(0)
    for j in range(_BN):
        s = idx_ref[g * _BN + j]
        w = w_ref[s].astype(jnp.bfloat16)
        x = x_ref[j].astype(jnp.bfloat16)
        o_ref[j] = jax.lax.dot_general(
            w, x,
            dimension_numbers=(((1,), (0,)), ((), ())),
            preferred_element_type=jnp.float32,
        )


def kernel(X, subject_idx, W):
    N, C, T = X.shape
    S = W.shape[0]

    grid_spec = pltpu.PrefetchScalarGridSpec(
        num_scalar_prefetch=1,
        grid=(N // _BN,),
        in_specs=[
            pl.BlockSpec((S, C, C), lambda n, idx: (0, 0, 0)),
            pl.BlockSpec((_BN, C, T), lambda n, idx: (n, 0, 0)),
        ],
        out_specs=pl.BlockSpec((_BN, C, T), lambda n, idx: (n, 0, 0)),
    )
    return pl.pallas_call(
        _body,
        grid_spec=grid_spec,
        out_shape=jax.ShapeDtypeStruct((N, C, T), jnp.float32),
    )(subject_idx, W, X)
